# Initial kernel scaffold; baseline (speedup 1.0000x reference)
import jax
import jax.numpy as jnp
from jax.experimental import pallas as pl


def kernel(pcl_noisy, pcl_clean, pcl_seeds, pcl_std, params):
    raise NotImplementedError("write your pallas kernel here")


# speedup vs baseline: 5.8999x; 5.8999x over previous
import functools

import jax
import jax.numpy as jnp
from jax import lax
from jax.experimental import pallas as pl
from jax.experimental.pallas import tpu as pltpu
from jax.experimental.pallas import tpu_sc as plsc

F32 = jnp.float32
BF16 = jnp.bfloat16
K_N = 32
KP1 = 33
NKEY = 10
NMOD = 4
DECAY = 4.0


def _leaky(x):
    return jnp.where(x >= 0, x, 0.2 * x)


def _bdot(a, b):
    return jnp.dot(a.astype(BF16), b.astype(BF16),
                   preferred_element_type=F32)


@functools.lru_cache(maxsize=None)
def _build_topk(B, N, C, RT):
    def body(xr_ref, xt_ref, a_ref, b_ref, dist_ref, idx_ref):
        xr = xr_ref[0]
        xt = xt_ref[0]
        a = a_ref[0]
        bb = b_ref[0]
        if C <= 4:
            e = xr[:, 0:1] * xt[0:1, :]
            for c in range(1, C):
                e = e + xr[:, c:c + 1] * xt[c:c + 1, :]
        else:
            e = _bdot(xr, xt)
        d = (a + bb) - 2.0 * e
        iot = lax.broadcasted_iota(jnp.int32, (RT, N), 1)
        dcols, icols = [], []
        dw = d
        for _ in range(KP1):
            v = jnp.min(dw, axis=1, keepdims=True)
            j = jnp.min(jnp.where(dw == v, iot, N), axis=1, keepdims=True)
            dcols.append(v)
            icols.append(j)
            dw = jnp.where(iot == j, jnp.inf, dw)
        dist_ref[0] = jnp.concatenate(dcols, axis=1)
        idx_ref[0] = jnp.concatenate(icols, axis=1)

    return pl.pallas_call(
        body,
        grid=(B, N // RT),
        in_specs=[pl.BlockSpec((1, RT, C), lambda b, t: (b, t, 0)),
                  pl.BlockSpec((1, C, N), lambda b, t: (b, 0, 0)),
                  pl.BlockSpec((1, RT, 1), lambda b, t: (b, t, 0)),
                  pl.BlockSpec((1, 1, N), lambda b, t: (b, 0, 0))],
        out_specs=[pl.BlockSpec((1, RT, KP1), lambda b, t: (b, t, 0)),
                   pl.BlockSpec((1, RT, KP1), lambda b, t: (b, t, 0))],
        out_shape=[jax.ShapeDtypeStruct((B, N, KP1), F32),
                   jax.ShapeDtypeStruct((B, N, KP1), jnp.int32)],
    )


@functools.lru_cache(maxsize=None)
def _build_mm(M, Cin, Cout, MT):
    def body(x_ref, w_ref, b_ref, y_ref):
        y_ref[...] = _bdot(x_ref[...], w_ref[...]) + b_ref[...]

    return pl.pallas_call(
        body,
        grid=(M // MT,),
        in_specs=[pl.BlockSpec((MT, Cin), lambda i: (i, 0)),
                  pl.BlockSpec((Cin, Cout), lambda i: (0, 0)),
                  pl.BlockSpec((1, Cout), lambda i: (0, 0))],
        out_specs=pl.BlockSpec((MT, Cout), lambda i: (i, 0)),
        out_shape=jax.ShapeDtypeStruct((M, Cout), F32),
    )


@functools.lru_cache(maxsize=None)
def _build_edge(M, C, RT):
    def body(x_ref, gx_ref, w_ref, b_ref, o_ref):
        x = x_ref[...]
        xb = jnp.broadcast_to(x[:, None, :], (RT, K_N, C))
        msg = jnp.concatenate([xb, gx_ref[...][:, :, :C] - xb], axis=2)
        mm = _bdot(msg.reshape(RT * K_N, 2 * C),
                   w_ref[...]).reshape(RT, K_N, 128)
        m = mm[:, 0, :]
        for k in range(1, K_N):
            m = jnp.maximum(m, mm[:, k, :])
        o_ref[...] = _leaky(m + b_ref[...])

    return pl.pallas_call(
        body,
        grid=(M // RT,),
        in_specs=[pl.BlockSpec((RT, C), lambda i: (i, 0)),
                  pl.BlockSpec((RT, K_N, 128), lambda i: (i, 0, 0)),
                  pl.BlockSpec((2 * C, 128), lambda i: (0, 0)),
                  pl.BlockSpec((1, 128), lambda i: (0, 0))],
        out_specs=pl.BlockSpec((RT, 128), lambda i: (i, 0)),
        out_shape=jax.ShapeDtypeStruct((M, 128), F32),
    )


@functools.lru_cache(maxsize=None)
def _build_feat(M3, RT):
    def body(zg_ref, d_ref, lw_ref, lb_ref, w_ref, cb_ref, f_ref):
        edb = d_ref[...].astype(BF16).astype(F32)
        distf = edb * lw_ref[...] + lb_ref[...]
        feat = jnp.concatenate([zg_ref[...][:, :64], distf], axis=1)
        f_ref[...] = _bdot(feat, w_ref[...]) + cb_ref[...]

    return pl.pallas_call(
        body,
        grid=(M3 // RT,),
        in_specs=[pl.BlockSpec((RT, 128), lambda i: (i, 0)),
                  pl.BlockSpec((RT, 1), lambda i: (i, 0)),
                  pl.BlockSpec((1, 64), lambda i: (0, 0)),
                  pl.BlockSpec((1, 64), lambda i: (0, 0)),
                  pl.BlockSpec((128, 128), lambda i: (0, 0)),
                  pl.BlockSpec((1, 128), lambda i: (0, 0))],
        out_specs=pl.BlockSpec((RT, 128), lambda i: (i, 0)),
        out_shape=jax.ShapeDtypeStruct((M3, 128), F32),
    )


@functools.lru_cache(maxsize=None)
def _build_fscore(M3, RT):
    def body(y_ref, o_ref):
        o_ref[...] = _leaky(jnp.max(y_ref[...], axis=1, keepdims=True))

    return pl.pallas_call(
        body,
        grid=(M3 // RT,),
        in_specs=[pl.BlockSpec((RT, 128), lambda i: (i, 0))],
        out_specs=pl.BlockSpec((RT, 1), lambda i: (i, 0)),
        out_shape=jax.ShapeDtypeStruct((M3, 1), F32),
    )


@functools.lru_cache(maxsize=None)
def _build_top10(M, RT):
    def body(s_ref, i_ref, o_ref):
        s = s_ref[...]
        iv = i_ref[...]
        iot = lax.broadcasted_iota(jnp.int32, (RT, KP1), 1)
        cols = []
        for _ in range(NKEY):
            vmax = jnp.max(s, axis=1, keepdims=True)
            pos = jnp.min(jnp.where(s == vmax, iot, KP1), axis=1,
                          keepdims=True)
            sel = jnp.sum(jnp.where(iot == pos, iv, 0), axis=1,
                          keepdims=True)
            cols.append(sel)
            s = jnp.where(iot == pos, -jnp.inf, s)
        o_ref[...] = jnp.concatenate(cols, axis=1)

    return pl.pallas_call(
        body,
        grid=(M // RT,),
        in_specs=[pl.BlockSpec((RT, KP1), lambda i: (i, 0)),
                  pl.BlockSpec((RT, KP1), lambda i: (i, 0))],
        out_specs=pl.BlockSpec((RT, NKEY), lambda i: (i, 0)),
        out_shape=jax.ShapeDtypeStruct((M, NKEY), jnp.int32),
    )


@functools.lru_cache(maxsize=None)
def _build_leaky_mm(M, Cin, Cout, MT):
    def body(h_ref, w_ref, b_ref, y_ref):
        y_ref[...] = _bdot(_leaky(h_ref[...]), w_ref[...]) + b_ref[...]

    return pl.pallas_call(
        body,
        grid=(M // MT,),
        in_specs=[pl.BlockSpec((MT, Cin), lambda i: (i, 0)),
                  pl.BlockSpec((Cin, Cout), lambda i: (0, 0)),
                  pl.BlockSpec((1, Cout), lambda i: (0, 0))],
        out_specs=pl.BlockSpec((MT, Cout), lambda i: (i, 0)),
        out_shape=jax.ShapeDtypeStruct((M, Cout), F32),
    )


@functools.lru_cache(maxsize=None)
def _build_head(M, Cin, MT):
    def body(h_ref, w1_ref, w2_ref, b2_ref, w3_ref, b3_ref, o_ref):
        z = _leaky(h_ref[...])
        a1 = jnp.maximum(_bdot(z, w1_ref[...]), 0.0)
        a2 = jnp.maximum(_bdot(a1, w2_ref[...]) + b2_ref[...], 0.0)
        o_ref[...] = _bdot(a2, w3_ref[...]) + b3_ref[...]

    return pl.pallas_call(
        body,
        grid=(M // MT,),
        in_specs=[pl.BlockSpec((MT, Cin), lambda i: (i, 0)),
                  pl.BlockSpec((512, 256), lambda i: (0, 0)),
                  pl.BlockSpec((256, 128), lambda i: (0, 0)),
                  pl.BlockSpec((1, 128), lambda i: (0, 0)),
                  pl.BlockSpec((128, 3), lambda i: (0, 0)),
                  pl.BlockSpec((1, 3), lambda i: (0, 0))],
        out_specs=pl.BlockSpec((MT, 3), lambda i: (i, 0)),
        out_shape=jax.ShapeDtypeStruct((M, 3), F32),
    )


@functools.lru_cache(maxsize=None)
def _build_sw(B, N):
    def body(x_ref, s_ref, o_ref):
        diff = x_ref[0] - s_ref[0]
        sd = diff[:, 0:1] * diff[:, 0:1]
        for c in range(1, 3):
            sd = sd + diff[:, c:c + 1] * diff[:, c:c + 1]
        iot = lax.broadcasted_iota(jnp.int32, (N, 1), 0)
        msd = jnp.sum(jnp.where(iot == N - 1, sd, 0.0))
        sdn = sd / (msd / 9.0)
        swv = jnp.exp(-sdn)
        o_ref[0] = swv / jnp.sum(swv)

    return pl.pallas_call(
        body,
        grid=(B,),
        in_specs=[pl.BlockSpec((1, N, 3), lambda b: (b, 0, 0)),
                  pl.BlockSpec((1, 1, 3), lambda b: (b, 0, 0))],
        out_specs=pl.BlockSpec((1, N, 1), lambda b: (b, 0, 0)),
        out_shape=jax.ShapeDtypeStruct((B, N, 1), F32),
    )


@functools.lru_cache(maxsize=None)
def _build_loss(B, N):
    def body(x_ref, t_ref, tt_ref, an_ref, tn_ref, p_ref, sw_ref, o_ref):
        x = x_ref[0]
        tt = tt_ref[0]
        a = an_ref[0]
        bb = tn_ref[0]
        e = x[:, 0:1] * tt[0:1, :]
        for c in range(1, 3):
            e = e + x[:, c:c + 1] * tt[c:c + 1, :]
        d = (a + bb) - 2.0 * e
        v = jnp.min(d, axis=1, keepdims=True)
        iot = lax.broadcasted_iota(jnp.int32, (N, N), 1)
        j = jnp.min(jnp.where(d == v, iot, N), axis=1, keepdims=True)
        oh = (iot == j).astype(F32)
        nb = jnp.concatenate(
            [jnp.sum(oh * tt[c:c + 1, :], axis=1, keepdims=True)
             for c in range(3)], axis=1)
        df = p_ref[0] - (nb - x)
        dist = df[:, 0:1] * df[:, 0:1]
        for c in range(1, 3):
            dist = dist + df[:, c:c + 1] * df[:, c:c + 1]
        o_ref[...] = jnp.reshape(jnp.sum(sw_ref[0] * dist), (1, 1, 1))

    return pl.pallas_call(
        body,
        grid=(B,),
        in_specs=[pl.BlockSpec((1, N, 3), lambda b: (b, 0, 0)),
                  pl.BlockSpec((1, N, 3), lambda b: (b, 0, 0)),
                  pl.BlockSpec((1, 3, N), lambda b: (b, 0, 0)),
                  pl.BlockSpec((1, N, 1), lambda b: (b, 0, 0)),
                  pl.BlockSpec((1, 1, N), lambda b: (b, 0, 0)),
                  pl.BlockSpec((1, N, 3), lambda b: (b, 0, 0)),
                  pl.BlockSpec((1, N, 1), lambda b: (b, 0, 0))],
        out_specs=pl.BlockSpec((1, 1, 1), lambda b: (b, 0, 0)),
        out_shape=jax.ShapeDtypeStruct((B, 1, 1), F32),
    )


_NW = 32


@functools.lru_cache(maxsize=None)
def _build_gather(V, D, Bt):
    bpw = Bt // _NW
    n = 1
    while True:
        if bpw % n == 0:
            c = bpw // n
            if c % 8 == 0 and c * (D + 1) * 4 <= 400_000:
                break
        n += 1
    chunk, nch = c, n
    mesh = plsc.VectorSubcoreMesh(core_axis_name="c", subcore_axis_name="s",
                                  num_cores=2, num_subcores=16)

    @functools.partial(
        pl.kernel, mesh=mesh,
        out_type=jax.ShapeDtypeStruct((Bt, D), F32),
        scratch_types=[pltpu.VMEM((chunk,), jnp.int32),
                       pltpu.VMEM((chunk, D), F32),
                       pltpu.SemaphoreType.DMA])
    def k(table_hbm, idx_hbm, out_hbm, idx_v, rows_v, sem):
        wid = lax.axis_index("s") * 2 + lax.axis_index("c")
        base = wid * bpw
        for i in range(nch):
            off = pl.multiple_of(base + i * chunk, 8)
            pltpu.sync_copy(idx_hbm.at[pl.ds(off, chunk)], idx_v)
            pltpu.async_copy(table_hbm.at[idx_v], rows_v, sem).wait()
            pltpu.sync_copy(rows_v, out_hbm.at[pl.ds(off, chunk)])

    return k


def _bn(x, gamma, beta, axes):
    mean = jnp.mean(x, axis=axes, keepdims=True)
    var = jnp.var(x, axis=axes, keepdims=True)
    return (x - mean) / jnp.sqrt(var + 1e-5) * gamma + beta


def _pad128(w):
    return jnp.pad(w, [(0, 0)] * (w.ndim - 1) + [(0, 128 - w.shape[-1])])


def _module_forward(p, x):
    B, N, _ = x.shape
    M = B * N
    off = jnp.arange(B, dtype=jnp.int32)[:, None, None] * N

    xT = jnp.transpose(x, (0, 2, 1))
    xn = jnp.sum(x ** 2, -1)
    dist, idx = _build_topk(B, N, 3, 256)(x, xT, xn[:, :, None],
                                          xn[:, None, :])
    idx_off = idx + off

    xf = x.reshape(M, 3)
    e_idx1 = idx_off[:, :, 1:].reshape(M * K_N)
    Gx1 = _build_gather(M, 128, M * K_N)(_pad128(xf), e_idx1)
    x1p = _build_edge(M, 3, 128)(
        xf, Gx1.reshape(M, K_N, 128), _pad128(p['c1W']),
        _pad128(p['c1b'])[None, :])
    x1 = x1p[:, :16]

    x1b = x1.reshape(B, N, 16)
    x1n = jnp.sum(x1b ** 2, -1)
    _, idx1 = _build_topk(B, N, 16, 256)(x1b, jnp.transpose(x1b, (0, 2, 1)),
                                         x1n[:, :, None], x1n[:, None, :])
    e_idx2 = (idx1 + off)[:, :, 1:].reshape(M * K_N)
    Gx2 = _build_gather(M, 128, M * K_N)(x1p, e_idx2)
    x2p = _build_edge(M, 16, 128)(
        x1, Gx2.reshape(M, K_N, 128), _pad128(p['c2W']),
        _pad128(p['c2b'])[None, :])
    x2 = x2p[:, :48]

    zp = _build_mm(M, 48, 128, 512)(x2, _pad128(p['l2W']),
                                    _pad128(p['l2b'])[None, :])
    f_idx = idx_off.reshape(M * KP1)
    Zg = _build_gather(M, 128, M * KP1)(zp, f_idx)
    ed2 = jnp.exp(-dist).reshape(M * KP1, 1)
    lw = p['l1W'][0:1].astype(BF16).astype(F32)
    f = _build_feat(M * KP1, 2048)(Zg, ed2, lw, p['l1b'][None, :],
                                   p['c3W'], p['c3b'][None, :])
    y = _bn(f.reshape(B, N, KP1, 128), p['c3g'], p['c3be'],
            axes=(0, 1, 2)).reshape(M * KP1, 128)
    score = _build_fscore(M * KP1, 2048)(y)
    top_idx = _build_top10(M, 512)(score.reshape(M, KP1),
                                   idx_off.reshape(M, KP1))

    G4 = _build_gather(M, 128, M * NKEY)(x2p, top_idx.reshape(M * NKEY))
    keyF = G4.reshape(M, NKEY, 128)[:, :, :48].reshape(M, NKEY * 48)
    h1 = _build_mm(M, NKEY * 48, 512, 256)(keyF, p['m1W1'],
                                           p['m1b1'][None, :])
    z1 = _bn(h1.reshape(B, N, 512), p['m1g1'], p['m1be1'],
             axes=(0, 1)).reshape(M, 512)
    h2 = _build_leaky_mm(M, 512, 512, 256)(z1, p['m1W2'],
                                           p['m1b2'][None, :])
    z2 = _bn(h2.reshape(B, N, 512), p['m1g2'], p['m1be2'],
             axes=(0, 1)).reshape(M, 512)
    pred = _build_head(M, 512, 256)(z2, p['dW1'], p['dW2'],
                                    p['db2'][None, :], p['dW3'],
                                    p['db3'][None, :])
    return jnp.tanh(pred).reshape(B, N, 3)


def kernel(pcl_noisy, pcl_clean, pcl_seeds, pcl_std, params):
    B, Nn, d = pcl_noisy.shape
    Nc = pcl_clean.shape[1]
    seeds_n = jnp.broadcast_to(pcl_seeds, (B, Nn, d))
    seeds_c = jnp.broadcast_to(pcl_seeds, (B, Nc, d))
    sw = _build_sw(B, Nn)(pcl_noisy, pcl_seeds)
    pn = pcl_noisy - seeds_n
    pc = pcl_clean - seeds_c
    nkey = jax.random.key(123)
    curr_std = pcl_std
    x = pn
    losses = []
    for i in range(NMOD):
        pred = _module_forward(params[i], x)
        if i < NMOD - 1:
            curr_std = curr_std / DECAY
            noise = jax.random.normal(jax.random.fold_in(nkey, i),
                                      pc.shape, dtype=F32)
            target = pc + noise * curr_std[:, None, None]
        else:
            target = pc
        xn = jnp.sum(x ** 2, -1)
        tn = jnp.sum(target ** 2, -1)
        lb = _build_loss(B, Nn)(x, target, jnp.transpose(target, (0, 2, 1)),
                                xn[:, :, None], tn[:, None, :],
                                pred, sw)
        losses.append(jnp.mean(lb))
        if i < NMOD - 1:
            x = x + pred
    return jnp.sum(jnp.stack(losses))
